# baseline (speedup 1.0000x reference)
import functools

import jax
import jax.numpy as jnp
from jax import lax
from jax.experimental import pallas as pl
from jax.experimental.pallas import tpu as pltpu
from jax.experimental.pallas import tpu_sc as plsc

_B = 16
_D = 128
_NC = 2
_NS = 16
_NW = _NC * _NS

_RPB = 128
_EB = 2500
_NB = 96
_EB_BASE = _EB // _NW
_EB_REM = _EB % _NW
_NB_PER_W = _NB // _NW


def _sc_body(edges_hbm, eids_hbm, nodes_hbm, nids_hbm,
             eout_hbm, nout_hbm,
             rows_v, ids_v, zero_v, eacc_s, nacc_s):
    c = lax.axis_index("c")
    s = lax.axis_index("s")
    wid = s * _NC + c

    @pl.when(s == 0)
    def _zero():
        for i in range(_B):
            for j in range(_D // 16):
                zero_v[i, pl.ds(j * 16, 16)] = jnp.zeros((16,), jnp.float32)
        pltpu.sync_copy(zero_v, eacc_s)
        pltpu.sync_copy(zero_v, nacc_s)

    plsc.subcore_barrier()

    e_start = wid * _EB_BASE + jnp.minimum(wid, _EB_REM)
    e_count = _EB_BASE + jnp.where(wid < _EB_REM, 1, 0)

    def _edge_step(j, carry):
        @pl.when(j < e_count)
        def _do():
            b = e_start + j
            pltpu.sync_copy(edges_hbm.at[pl.ds(b * _RPB, _RPB)], rows_v)
            pltpu.sync_copy(eids_hbm.at[b], ids_v)
            pltpu.sync_copy(rows_v, eacc_s.at[ids_v], add=True)
        return carry

    lax.fori_loop(0, _EB_BASE + 1, _edge_step, 0)

    n_start = wid * _NB_PER_W

    def _node_step(j, carry):
        b = n_start + j
        pltpu.sync_copy(nodes_hbm.at[pl.ds(b * _RPB, _RPB)], rows_v)
        pltpu.sync_copy(nids_hbm.at[b], ids_v)
        pltpu.sync_copy(rows_v, nacc_s.at[ids_v], add=True)
        return carry

    lax.fori_loop(0, _NB_PER_W, _node_step, 0)

    plsc.subcore_barrier()

    @pl.when(s == 0)
    def _drain():
        pltpu.sync_copy(eacc_s, eout_hbm.at[c])
        pltpu.sync_copy(nacc_s, nout_hbm.at[c])


def _sc_segment_sums(edges, eids2, nodes_p, nids2):
    mesh = plsc.VectorSubcoreMesh(core_axis_name="c", subcore_axis_name="s")
    f = pl.kernel(
        _sc_body,
        out_type=[
            jax.ShapeDtypeStruct((_NC, _B, _D), jnp.float32),
            jax.ShapeDtypeStruct((_NC, _B, _D), jnp.float32),
        ],
        mesh=mesh,
        scratch_types=[
            pltpu.VMEM((_RPB, _D), jnp.float32),
            pltpu.VMEM((_RPB,), jnp.int32),
            pltpu.VMEM((_B, _D), jnp.float32),
            pltpu.VMEM_SHARED((_B, _D), jnp.float32),
            pltpu.VMEM_SHARED((_B, _D), jnp.float32),
        ],
    )
    return f(edges, eids2, nodes_p, nids2)


def _mlp_body(g_ref, n_ref, e_ref, w1_ref, b1_ref, w2_ref, b2_ref,
              gm_ref, bt_ref, o_ref):
    n_agg = n_ref[0] + n_ref[1]
    e_agg = e_ref[0] + e_ref[1]
    h = (
        jnp.dot(g_ref[...], w1_ref[0:_D, :], preferred_element_type=jnp.float32)
        + jnp.dot(n_agg, w1_ref[_D : 2 * _D, :], preferred_element_type=jnp.float32)
        + jnp.dot(e_agg, w1_ref[2 * _D : 3 * _D, :], preferred_element_type=jnp.float32)
        + b1_ref[...]
    )
    h = jnp.maximum(h, 0.0)
    y = jnp.dot(h, w2_ref[...], preferred_element_type=jnp.float32) + b2_ref[...]
    out = 1.0 / (1.0 + jnp.exp(-y))
    mean = jnp.mean(out, axis=-1, keepdims=True)
    ctr = out - mean
    var = jnp.mean(ctr * ctr, axis=-1, keepdims=True)
    normed = ctr * lax.rsqrt(var + 1e-3)
    o_ref[...] = normed * gm_ref[...] + bt_ref[...]


def _mlp(globals_feat, nparts, eparts, W1, b1, W2, b2, gamma, beta):
    return pl.pallas_call(
        _mlp_body,
        out_shape=jax.ShapeDtypeStruct((_B, _D), jnp.float32),
    )(
        globals_feat,
        nparts,
        eparts,
        W1,
        b1.reshape(1, -1),
        W2,
        b2.reshape(1, -1),
        gamma.reshape(1, -1),
        beta.reshape(1, -1),
    )


def kernel(globals_feat, nodes, edges, node_segment_ids, edge_segment_ids,
           W1, b1, W2, b2, gamma, beta):
    n_pad = _NB * _RPB
    nodes_p = jnp.zeros((n_pad, _D), jnp.float32).at[: nodes.shape[0]].set(nodes)
    nids_p = jnp.zeros((n_pad,), jnp.int32).at[: node_segment_ids.shape[0]].set(
        node_segment_ids
    )
    eids2 = edge_segment_ids.reshape(_EB, _RPB)
    nids2 = nids_p.reshape(_NB, _RPB)
    eparts, nparts = _sc_segment_sums(edges, eids2, nodes_p, nids2)
    return _mlp(globals_feat, nparts, eparts, W1, b1, W2, b2, gamma, beta)


# speedup vs baseline: 1.8144x; 1.8144x over previous
import functools

import jax
import jax.numpy as jnp
from jax import lax
from jax.experimental import pallas as pl
from jax.experimental.pallas import tpu as pltpu
from jax.experimental.pallas import tpu_sc as plsc

_B = 16
_D = 128
_NC = 2
_NS = 16
_NW = _NC * _NS

_RPB = 128
_EB = 2500
_NB = 96
_EB_MAIN = _EB // _NW
_EB_TAIL = _EB - _EB_MAIN * _NW
_NB_PER_W = _NB // _NW


def _sc_body(edges_hbm, eids_hbm, nodes_hbm, nids_hbm,
             eout_hbm, nout_hbm,
             buf0, buf1, idb0, idb1, zero_v, eacc_s, nacc_s,
             sg0, sg1, si0, si1, ss0, ss1):
    c = lax.axis_index("c")
    s = lax.axis_index("s")
    wid = s * _NC + c

    @pl.when(s == 0)
    def _zero():
        for i in range(_B):
            for j in range(_D // 16):
                zero_v[i, pl.ds(j * 16, 16)] = jnp.zeros((16,), jnp.float32)
        pltpu.sync_copy(zero_v, eacc_s)
        pltpu.sync_copy(zero_v, nacc_s)

    plsc.subcore_barrier()

    n_start = wid * _NB_PER_W
    for j in range(_NB_PER_W):
        pltpu.sync_copy(nids_hbm.at[n_start + j], idb0)
        pltpu.sync_copy(nodes_hbm.at[pl.ds((n_start + j) * _RPB, _RPB)], buf0)
        pltpu.sync_copy(buf0, nacc_s.at[idb0], add=True)

    @pl.when(wid < _EB_TAIL)
    def _tail():
        b = _EB_MAIN * _NW + wid
        pltpu.sync_copy(eids_hbm.at[b], idb0)
        pltpu.sync_copy(edges_hbm.at[pl.ds(b * _RPB, _RPB)], buf0)
        pltpu.sync_copy(buf0, eacc_s.at[idb0], add=True)

    base = wid * _EB_MAIN

    def _gather(j, buf, idb, sd, si):
        b = base + j
        pltpu.async_copy(edges_hbm.at[pl.ds(b * _RPB, _RPB)], buf, sd)
        pltpu.async_copy(eids_hbm.at[b], idb, si)

    def _wait_gather(buf, idb, sd, si):
        pltpu.make_async_copy(edges_hbm.at[pl.ds(0, _RPB)], buf, sd).wait()
        pltpu.make_async_copy(eids_hbm.at[0], idb, si).wait()

    def _scatter(buf, idb, sem):
        pltpu.async_copy(buf, eacc_s.at[idb], sem, add=True)

    def _wait_scatter(buf, idb, sem):
        pltpu.make_async_copy(buf, eacc_s.at[idb], sem).wait()

    _gather(0, buf0, idb0, sg0, si0)
    _gather(1, buf1, idb1, sg1, si1)
    _wait_gather(buf0, idb0, sg0, si0)
    _scatter(buf0, idb0, ss0)

    def _pair(i, carry):
        j0 = 2 * i
        _wait_scatter(buf0, idb0, ss0)
        _gather(j0, buf0, idb0, sg0, si0)
        _wait_gather(buf1, idb1, sg1, si1)
        _scatter(buf1, idb1, ss1)
        _wait_scatter(buf1, idb1, ss1)
        _gather(j0 + 1, buf1, idb1, sg1, si1)
        _wait_gather(buf0, idb0, sg0, si0)
        _scatter(buf0, idb0, ss0)
        return carry

    lax.fori_loop(1, _EB_MAIN // 2, _pair, 0)

    _wait_gather(buf1, idb1, sg1, si1)
    _scatter(buf1, idb1, ss1)
    _wait_scatter(buf0, idb0, ss0)
    _wait_scatter(buf1, idb1, ss1)

    plsc.subcore_barrier()

    @pl.when(s == 0)
    def _drain():
        pltpu.sync_copy(eacc_s, eout_hbm.at[c])
        pltpu.sync_copy(nacc_s, nout_hbm.at[c])


def _sc_segment_sums(edges, eids2, nodes_p, nids2):
    mesh = plsc.VectorSubcoreMesh(core_axis_name="c", subcore_axis_name="s")
    f = pl.kernel(
        _sc_body,
        out_type=[
            jax.ShapeDtypeStruct((_NC, _B, _D), jnp.float32),
            jax.ShapeDtypeStruct((_NC, _B, _D), jnp.float32),
        ],
        mesh=mesh,
        scratch_types=[
            pltpu.VMEM((_RPB, _D), jnp.float32),
            pltpu.VMEM((_RPB, _D), jnp.float32),
            pltpu.VMEM((_RPB,), jnp.int32),
            pltpu.VMEM((_RPB,), jnp.int32),
            pltpu.VMEM((_B, _D), jnp.float32),
            pltpu.VMEM_SHARED((_B, _D), jnp.float32),
            pltpu.VMEM_SHARED((_B, _D), jnp.float32),
            pltpu.SemaphoreType.DMA,
            pltpu.SemaphoreType.DMA,
            pltpu.SemaphoreType.DMA,
            pltpu.SemaphoreType.DMA,
            pltpu.SemaphoreType.DMA,
            pltpu.SemaphoreType.DMA,
        ],
    )
    return f(edges, eids2, nodes_p, nids2)


def _mlp_body(g_ref, n_ref, e_ref, w1_ref, b1_ref, w2_ref, b2_ref,
              gm_ref, bt_ref, o_ref):
    n_agg = n_ref[0] + n_ref[1]
    e_agg = e_ref[0] + e_ref[1]
    h = (
        jnp.dot(g_ref[...], w1_ref[0:_D, :], preferred_element_type=jnp.float32)
        + jnp.dot(n_agg, w1_ref[_D : 2 * _D, :], preferred_element_type=jnp.float32)
        + jnp.dot(e_agg, w1_ref[2 * _D : 3 * _D, :], preferred_element_type=jnp.float32)
        + b1_ref[...]
    )
    h = jnp.maximum(h, 0.0)
    y = jnp.dot(h, w2_ref[...], preferred_element_type=jnp.float32) + b2_ref[...]
    out = 1.0 / (1.0 + jnp.exp(-y))
    mean = jnp.mean(out, axis=-1, keepdims=True)
    ctr = out - mean
    var = jnp.mean(ctr * ctr, axis=-1, keepdims=True)
    normed = ctr * lax.rsqrt(var + 1e-3)
    o_ref[...] = normed * gm_ref[...] + bt_ref[...]


def _mlp(globals_feat, nparts, eparts, W1, b1, W2, b2, gamma, beta):
    return pl.pallas_call(
        _mlp_body,
        out_shape=jax.ShapeDtypeStruct((_B, _D), jnp.float32),
    )(
        globals_feat,
        nparts,
        eparts,
        W1,
        b1.reshape(1, -1),
        W2,
        b2.reshape(1, -1),
        gamma.reshape(1, -1),
        beta.reshape(1, -1),
    )


def kernel(globals_feat, nodes, edges, node_segment_ids, edge_segment_ids,
           W1, b1, W2, b2, gamma, beta):
    n_pad = _NB * _RPB
    nodes_p = jnp.zeros((n_pad, _D), jnp.float32).at[: nodes.shape[0]].set(nodes)
    nids_p = jnp.zeros((n_pad,), jnp.int32).at[: node_segment_ids.shape[0]].set(
        node_segment_ids
    )
    eids2 = edge_segment_ids.reshape(_EB, _RPB)
    nids2 = nids_p.reshape(_NB, _RPB)
    eparts, nparts = _sc_segment_sums(edges, eids2, nodes_p, nids2)
    return _mlp(globals_feat, nparts, eparts, W1, b1, W2, b2, gamma, beta)


# speedup vs baseline: 1.9124x; 1.0540x over previous
import functools

import jax
import jax.numpy as jnp
from jax import lax
from jax.experimental import pallas as pl
from jax.experimental.pallas import tpu as pltpu
from jax.experimental.pallas import tpu_sc as plsc

_B = 16
_D = 128
_NC = 2
_NS = 16
_NW = _NC * _NS

_RPB = 128
_EB = 2500
_NB = 96
_EB_MAIN = _EB // _NW
_EB_TAIL = _EB - _EB_MAIN * _NW
_NB_PER_W = _NB // _NW
_L = 16
_NCH = _D // _L


def _reduce_batch(buf, idb, acc_v, acc_s):
    idv_first = idb[pl.ds(0, _L)]
    idv_last = idb[pl.ds(_RPB - _L, _L)]
    m = idv_first[0]
    uniform = m == idv_last[_L - 1]

    @pl.when(uniform)
    def _fast():
        zero = jnp.zeros((_L,), jnp.float32)

        def _rows(r4, carry):
            accs = list(carry)
            for dr in range(4):
                r = r4 * 4 + dr
                for k in range(_NCH):
                    accs[k] = accs[k] + buf[r, pl.ds(_L * k, _L)]
            return tuple(accs)

        accs = lax.fori_loop(0, _RPB // 4, _rows, (zero,) * _NCH)
        for k in range(_NCH):
            acc_v[m, pl.ds(_L * k, _L)] = acc_v[m, pl.ds(_L * k, _L)] + accs[k]

    @pl.when(jnp.logical_not(uniform))
    def _slow():
        pltpu.sync_copy(buf, acc_s.at[idb], add=True)


def _sc_body(edges_hbm, eids_hbm, nodes_hbm, nids_hbm,
             eout_hbm, nout_hbm,
             buf0, buf1, idb0, idb1, eacc_v, nacc_v, zero_v, eacc_s, nacc_s,
             sg0, sg1, si0, si1):
    c = lax.axis_index("c")
    s = lax.axis_index("s")
    wid = s * _NC + c

    for i in range(_B):
        for j in range(_NCH):
            eacc_v[i, pl.ds(j * _L, _L)] = jnp.zeros((_L,), jnp.float32)
            nacc_v[i, pl.ds(j * _L, _L)] = jnp.zeros((_L,), jnp.float32)
            zero_v[i, pl.ds(j * _L, _L)] = jnp.zeros((_L,), jnp.float32)

    @pl.when(s == 0)
    def _zero_shared():
        pltpu.sync_copy(zero_v, eacc_s)
        pltpu.sync_copy(zero_v, nacc_s)

    plsc.subcore_barrier()

    n_start = wid * _NB_PER_W
    for j in range(_NB_PER_W):
        pltpu.sync_copy(nids_hbm.at[n_start + j], idb0)
        pltpu.sync_copy(nodes_hbm.at[pl.ds((n_start + j) * _RPB, _RPB)], buf0)
        _reduce_batch(buf0, idb0, nacc_v, nacc_s)

    @pl.when(wid < _EB_TAIL)
    def _tail():
        b = _EB_MAIN * _NW + wid
        pltpu.sync_copy(eids_hbm.at[b], idb0)
        pltpu.sync_copy(edges_hbm.at[pl.ds(b * _RPB, _RPB)], buf0)
        _reduce_batch(buf0, idb0, eacc_v, eacc_s)

    base = wid * _EB_MAIN

    def _gather(j, buf, idb, sd, si):
        b = base + j
        pltpu.async_copy(edges_hbm.at[pl.ds(b * _RPB, _RPB)], buf, sd)
        pltpu.async_copy(eids_hbm.at[b], idb, si)

    def _wait_gather(buf, idb, sd, si):
        pltpu.make_async_copy(edges_hbm.at[pl.ds(0, _RPB)], buf, sd).wait()
        pltpu.make_async_copy(eids_hbm.at[0], idb, si).wait()

    _gather(0, buf0, idb0, sg0, si0)

    def _pair(i, carry):
        j0 = 2 * i
        _gather(j0 + 1, buf1, idb1, sg1, si1)
        _wait_gather(buf0, idb0, sg0, si0)
        _reduce_batch(buf0, idb0, eacc_v, eacc_s)

        @pl.when(j0 + 2 < _EB_MAIN)
        def _next_even():
            _gather(j0 + 2, buf0, idb0, sg0, si0)

        _wait_gather(buf1, idb1, sg1, si1)
        _reduce_batch(buf1, idb1, eacc_v, eacc_s)
        return carry

    lax.fori_loop(0, _EB_MAIN // 2, _pair, 0)

    plsc.subcore_barrier()

    pltpu.sync_copy(eacc_v, eout_hbm.at[c].at[s])
    pltpu.sync_copy(nacc_v, nout_hbm.at[c].at[s])

    @pl.when(s == 0)
    def _drain_shared():
        pltpu.sync_copy(eacc_s, eout_hbm.at[c].at[_NS])
        pltpu.sync_copy(nacc_s, nout_hbm.at[c].at[_NS])


def _sc_segment_sums(edges, eids2, nodes_p, nids2):
    mesh = plsc.VectorSubcoreMesh(core_axis_name="c", subcore_axis_name="s")
    f = pl.kernel(
        _sc_body,
        out_type=[
            jax.ShapeDtypeStruct((_NC, _NS + 1, _B, _D), jnp.float32),
            jax.ShapeDtypeStruct((_NC, _NS + 1, _B, _D), jnp.float32),
        ],
        mesh=mesh,
        scratch_types=[
            pltpu.VMEM((_RPB, _D), jnp.float32),
            pltpu.VMEM((_RPB, _D), jnp.float32),
            pltpu.VMEM((_RPB,), jnp.int32),
            pltpu.VMEM((_RPB,), jnp.int32),
            pltpu.VMEM((_B, _D), jnp.float32),
            pltpu.VMEM((_B, _D), jnp.float32),
            pltpu.VMEM((_B, _D), jnp.float32),
            pltpu.VMEM_SHARED((_B, _D), jnp.float32),
            pltpu.VMEM_SHARED((_B, _D), jnp.float32),
            pltpu.SemaphoreType.DMA,
            pltpu.SemaphoreType.DMA,
            pltpu.SemaphoreType.DMA,
            pltpu.SemaphoreType.DMA,
        ],
    )
    return f(edges, eids2, nodes_p, nids2)


def _mlp_body(g_ref, n_ref, e_ref, w1_ref, b1_ref, w2_ref, b2_ref,
              gm_ref, bt_ref, o_ref):
    n_agg = jnp.sum(n_ref[...], axis=0)
    e_agg = jnp.sum(e_ref[...], axis=0)
    h = (
        jnp.dot(g_ref[...], w1_ref[0:_D, :], preferred_element_type=jnp.float32)
        + jnp.dot(n_agg, w1_ref[_D : 2 * _D, :], preferred_element_type=jnp.float32)
        + jnp.dot(e_agg, w1_ref[2 * _D : 3 * _D, :], preferred_element_type=jnp.float32)
        + b1_ref[...]
    )
    h = jnp.maximum(h, 0.0)
    y = jnp.dot(h, w2_ref[...], preferred_element_type=jnp.float32) + b2_ref[...]
    out = 1.0 / (1.0 + jnp.exp(-y))
    mean = jnp.mean(out, axis=-1, keepdims=True)
    ctr = out - mean
    var = jnp.mean(ctr * ctr, axis=-1, keepdims=True)
    normed = ctr * lax.rsqrt(var + 1e-3)
    o_ref[...] = normed * gm_ref[...] + bt_ref[...]


def _mlp(globals_feat, nparts, eparts, W1, b1, W2, b2, gamma, beta):
    return pl.pallas_call(
        _mlp_body,
        out_shape=jax.ShapeDtypeStruct((_B, _D), jnp.float32),
    )(
        globals_feat,
        nparts,
        eparts,
        W1,
        b1.reshape(1, -1),
        W2,
        b2.reshape(1, -1),
        gamma.reshape(1, -1),
        beta.reshape(1, -1),
    )


def kernel(globals_feat, nodes, edges, node_segment_ids, edge_segment_ids,
           W1, b1, W2, b2, gamma, beta):
    n_pad = _NB * _RPB
    nodes_p = jnp.zeros((n_pad, _D), jnp.float32).at[: nodes.shape[0]].set(nodes)
    nids_p = jnp.full((n_pad,), _B - 1, jnp.int32).at[
        : node_segment_ids.shape[0]
    ].set(node_segment_ids)
    eids2 = edge_segment_ids.reshape(_EB, _RPB)
    nids2 = nids_p.reshape(_NB, _RPB)
    eparts, nparts = _sc_segment_sums(edges, eids2, nodes_p, nids2)
    eparts = eparts.reshape(_NC * (_NS + 1), _B, _D)
    nparts = nparts.reshape(_NC * (_NS + 1), _B, _D)
    return _mlp(globals_feat, nparts, eparts, W1, b1, W2, b2, gamma, beta)


# speedup vs baseline: 2.1758x; 1.1378x over previous
import functools

import jax
import jax.numpy as jnp
from jax import lax
from jax.experimental import pallas as pl
from jax.experimental.pallas import tpu as pltpu
from jax.experimental.pallas import tpu_sc as plsc

_B = 16
_D = 128
_NC = 2
_NS = 16
_NW = _NC * _NS

_CH = 128
_RPG = 256
_NE = 320000
_NN = 10000
_EG = _NE // _RPG
_EG_MAIN = _EG // _NW
_EG_TAIL = _EG - _EG_MAIN * _NW
_NG = _NN // _RPG
_NN_TAIL = _NN - _NG * _RPG
_L = 16
_NCH = _D // _L


def _reduce_chunk(buf, idb, acc_v, acc_s, off):
    idv_first = idb[pl.ds(0, _L)]
    idv_last = idb[pl.ds(_CH - _L, _L)]
    m = idv_first[0]
    uniform = m == idv_last[_L - 1]

    @pl.when(uniform)
    def _fast():
        zero = jnp.zeros((_L,), jnp.float32)

        def _rows(r4, carry):
            accs = list(carry)
            for dr in range(4):
                r = off + r4 * 4 + dr
                for k in range(_NCH):
                    accs[k] = accs[k] + buf[r, pl.ds(_L * k, _L)]
            return tuple(accs)

        accs = lax.fori_loop(0, _CH // 4, _rows, (zero,) * _NCH)
        for k in range(_NCH):
            acc_v[m, pl.ds(_L * k, _L)] = acc_v[m, pl.ds(_L * k, _L)] + accs[k]

    @pl.when(jnp.logical_not(uniform))
    def _slow():
        pltpu.sync_copy(buf.at[pl.ds(off, _CH)], acc_s.at[idb], add=True)


def _sc_body(edges_hbm, eids_hbm, nodes_hbm, nids_hbm,
             eout_hbm, nout_hbm,
             buf0, buf1, id0a, id0b, id1a, id1b, idt, zero_v, eacc_s, nacc_s,
             sg0, sg1, si0, si1):
    c = lax.axis_index("c")
    s = lax.axis_index("s")
    wid = s * _NC + c
    eacc_v = zero_v
    del zero_v

    for i in range(_B):
        for j in range(_NCH):
            eacc_v[i, pl.ds(j * _L, _L)] = jnp.zeros((_L,), jnp.float32)
            buf0[i, pl.ds(j * _L, _L)] = jnp.zeros((_L,), jnp.float32)

    @pl.when(s == 0)
    def _zero_shared():
        pltpu.sync_copy(buf0.at[pl.ds(0, _B)], eacc_s)
        pltpu.sync_copy(buf0.at[pl.ds(0, _B)], nacc_s)

    plsc.subcore_barrier()

    def _gather_into(src_hbm, ids_hbm, g, buf, ida, idb_, sd, si):
        r0 = g * _RPG
        pltpu.async_copy(src_hbm.at[pl.ds(r0, _RPG)], buf, sd)
        pltpu.async_copy(ids_hbm.at[pl.ds(r0, _CH)], ida, si)
        pltpu.async_copy(ids_hbm.at[pl.ds(r0 + _CH, _CH)], idb_, si)

    def _wait_gather(src_hbm, ids_hbm, buf, ida, idb_, sd, si):
        pltpu.make_async_copy(src_hbm.at[pl.ds(0, _RPG)], buf, sd).wait()
        pltpu.make_async_copy(ids_hbm.at[pl.ds(0, _CH)], ida, si).wait()
        pltpu.make_async_copy(ids_hbm.at[pl.ds(0, _CH)], idb_, si).wait()

    def _reduce_gather(buf, ida, idb_, acc_v, acc_s):
        _reduce_chunk(buf, ida, acc_v, acc_s, 0)
        _reduce_chunk(buf, idb_, acc_v, acc_s, _CH)

    ebase = wid * _EG_MAIN

    def _e_gather(j, buf, ida, idb_, sd, si):
        _gather_into(edges_hbm, eids_hbm, ebase + j, buf, ida, idb_, sd, si)

    _e_gather(0, buf0, id0a, id0b, sg0, si0)

    def _pair(i, carry):
        j0 = 2 * i
        _e_gather(j0 + 1, buf1, id1a, id1b, sg1, si1)
        _wait_gather(edges_hbm, eids_hbm, buf0, id0a, id0b, sg0, si0)
        _reduce_gather(buf0, id0a, id0b, eacc_v, eacc_s)

        @pl.when(j0 + 2 < _EG_MAIN)
        def _next_even():
            _e_gather(j0 + 2, buf0, id0a, id0b, sg0, si0)

        _wait_gather(edges_hbm, eids_hbm, buf1, id1a, id1b, sg1, si1)
        _reduce_gather(buf1, id1a, id1b, eacc_v, eacc_s)
        return carry

    lax.fori_loop(0, _EG_MAIN // 2, _pair, 0)

    _wait_gather(edges_hbm, eids_hbm, buf0, id0a, id0b, sg0, si0)
    _reduce_gather(buf0, id0a, id0b, eacc_v, eacc_s)

    @pl.when(wid < _EG_TAIL)
    def _etail():
        g = _EG_MAIN * _NW + wid
        _e_gather(g - ebase, buf0, id0a, id0b, sg0, si0)
        _wait_gather(edges_hbm, eids_hbm, buf0, id0a, id0b, sg0, si0)
        _reduce_gather(buf0, id0a, id0b, eacc_v, eacc_s)

    pltpu.sync_copy(eacc_v, eout_hbm.at[c].at[s])

    for i in range(_B):
        for j in range(_NCH):
            eacc_v[i, pl.ds(j * _L, _L)] = jnp.zeros((_L,), jnp.float32)

    for rnd in range(2):
        g = wid + rnd * _NW

        @pl.when(g < _NG)
        def _node_round():
            _gather_into(nodes_hbm, nids_hbm, g, buf0, id0a, id0b, sg0, si0)
            _wait_gather(nodes_hbm, nids_hbm, buf0, id0a, id0b, sg0, si0)
            _reduce_gather(buf0, id0a, id0b, eacc_v, nacc_s)

    @pl.when(wid == 7)
    def _ntail():
        r0 = _NG * _RPG
        pltpu.sync_copy(nids_hbm.at[pl.ds(r0, _NN_TAIL)], idt)
        pltpu.sync_copy(nodes_hbm.at[pl.ds(r0, _NN_TAIL)], buf0.at[pl.ds(0, _NN_TAIL)])
        pltpu.sync_copy(buf0.at[pl.ds(0, _NN_TAIL)], nacc_s.at[idt], add=True)

    plsc.subcore_barrier()

    pltpu.sync_copy(eacc_v, nout_hbm.at[c].at[s])

    @pl.when(s == 0)
    def _drain_shared():
        pltpu.sync_copy(eacc_s, eout_hbm.at[c].at[_NS])
        pltpu.sync_copy(nacc_s, nout_hbm.at[c].at[_NS])


def _sc_segment_sums(edges, eids, nodes, nids):
    mesh = plsc.VectorSubcoreMesh(core_axis_name="c", subcore_axis_name="s")
    f = pl.kernel(
        _sc_body,
        out_type=[
            jax.ShapeDtypeStruct((_NC, _NS + 1, _B, _D), jnp.float32),
            jax.ShapeDtypeStruct((_NC, _NS + 1, _B, _D), jnp.float32),
        ],
        mesh=mesh,
        scratch_types=[
            pltpu.VMEM((_RPG, _D), jnp.float32),
            pltpu.VMEM((_RPG, _D), jnp.float32),
            pltpu.VMEM((_CH,), jnp.int32),
            pltpu.VMEM((_CH,), jnp.int32),
            pltpu.VMEM((_CH,), jnp.int32),
            pltpu.VMEM((_CH,), jnp.int32),
            pltpu.VMEM((_NN_TAIL,), jnp.int32),
            pltpu.VMEM((_B, _D), jnp.float32),
            pltpu.VMEM_SHARED((_B, _D), jnp.float32),
            pltpu.VMEM_SHARED((_B, _D), jnp.float32),
            pltpu.SemaphoreType.DMA,
            pltpu.SemaphoreType.DMA,
            pltpu.SemaphoreType.DMA,
            pltpu.SemaphoreType.DMA,
        ],
    )
    return f(edges, eids, nodes, nids)


def _mlp_body(g_ref, n_ref, e_ref, w1_ref, b1_ref, w2_ref, b2_ref,
              gm_ref, bt_ref, o_ref):
    n_agg = jnp.sum(n_ref[...], axis=0)
    e_agg = jnp.sum(e_ref[...], axis=0)
    h = (
        jnp.dot(g_ref[...], w1_ref[0:_D, :], preferred_element_type=jnp.float32)
        + jnp.dot(n_agg, w1_ref[_D : 2 * _D, :], preferred_element_type=jnp.float32)
        + jnp.dot(e_agg, w1_ref[2 * _D : 3 * _D, :], preferred_element_type=jnp.float32)
        + b1_ref[...]
    )
    h = jnp.maximum(h, 0.0)
    y = jnp.dot(h, w2_ref[...], preferred_element_type=jnp.float32) + b2_ref[...]
    out = 1.0 / (1.0 + jnp.exp(-y))
    mean = jnp.mean(out, axis=-1, keepdims=True)
    ctr = out - mean
    var = jnp.mean(ctr * ctr, axis=-1, keepdims=True)
    normed = ctr * lax.rsqrt(var + 1e-3)
    o_ref[...] = normed * gm_ref[...] + bt_ref[...]


def _mlp(globals_feat, nparts, eparts, W1, b1, W2, b2, gamma, beta):
    return pl.pallas_call(
        _mlp_body,
        out_shape=jax.ShapeDtypeStruct((_B, _D), jnp.float32),
    )(
        globals_feat,
        nparts,
        eparts,
        W1,
        b1.reshape(1, -1),
        W2,
        b2.reshape(1, -1),
        gamma.reshape(1, -1),
        beta.reshape(1, -1),
    )


def kernel(globals_feat, nodes, edges, node_segment_ids, edge_segment_ids,
           W1, b1, W2, b2, gamma, beta):
    eparts, nparts = _sc_segment_sums(
        edges, edge_segment_ids, nodes, node_segment_ids
    )
    eparts = eparts.reshape(_NC * (_NS + 1), _B, _D)
    nparts = nparts.reshape(_NC * (_NS + 1), _B, _D)
    return _mlp(globals_feat, nparts, eparts, W1, b1, W2, b2, gamma, beta)
